# baseline (speedup 1.0000x reference)
import functools

import jax
import jax.numpy as jnp
from jax.experimental import pallas as pl
from jax.experimental.pallas import tpu as pltpu


def _mlp_kernel(x_ref, w1_ref, b1_ref, w2_ref, b2_ref, o_ref):
    x = x_ref[...].astype(jnp.bfloat16)
    w1 = w1_ref[...].astype(jnp.bfloat16)
    h = jnp.dot(x, w1, preferred_element_type=jnp.float32)
    h = jnp.maximum(h + b1_ref[...], 0.0).astype(jnp.bfloat16)
    w2 = w2_ref[...].astype(jnp.bfloat16)
    q = jnp.dot(h, w2, preferred_element_type=jnp.float32) + b2_ref[...]
    o_ref[...] = q[:, : o_ref.shape[-1]].astype(o_ref.dtype)


@functools.partial(jax.jit, static_argnames=("num_actions", "tb"))
def _forward(x, w1p, b1p, w2p, b2p, *, num_actions, tb):
    B, K = x.shape
    H = w1p.shape[1]
    Ap = w2p.shape[1]
    A = num_actions

    grid = (pl.cdiv(B, tb),)
    cost = pl.CostEstimate(
        flops=2 * B * (K * H + H * Ap),
        transcendentals=0,
        bytes_accessed=4 * (B * K + B * A + K * H + H * Ap + H + Ap),
    )

    return pl.pallas_call(
        _mlp_kernel,
        out_shape=jax.ShapeDtypeStruct((B, A), x.dtype),
        grid=grid,
        in_specs=[
            pl.BlockSpec((tb, K), lambda i: (i, 0)),
            pl.BlockSpec((K, H), lambda i: (0, 0)),
            pl.BlockSpec((1, H), lambda i: (0, 0)),
            pl.BlockSpec((H, Ap), lambda i: (0, 0)),
            pl.BlockSpec((1, Ap), lambda i: (0, 0)),
        ],
        out_specs=pl.BlockSpec((tb, A), lambda i: (i, 0)),
        compiler_params=pltpu.CompilerParams(
            dimension_semantics=("parallel",),
            vmem_limit_bytes=96 * 1024 * 1024,
        ),
        cost_estimate=cost,
    )(x, w1p, b1p, w2p, b2p)


def kernel(x, w1p, b1p, w2p, b2p):
    return _forward(x, w1p, b1p, w2p, b2p, num_actions=18, tb=4096)


# speedup vs baseline: 1.0197x; 1.0197x over previous
import functools

import jax
import jax.numpy as jnp
from jax.experimental import pallas as pl
from jax.experimental.pallas import tpu as pltpu


def _mlp_kernel(x_ref, w1_ref, b1_ref, w2_ref, b2_ref, o_ref):
    x = x_ref[...].astype(jnp.bfloat16)
    w1 = w1_ref[...].astype(jnp.bfloat16)
    h = jnp.dot(x, w1, preferred_element_type=jnp.float32)
    h = jnp.maximum(h + b1_ref[...], 0.0).astype(jnp.bfloat16)
    w2 = w2_ref[...].astype(jnp.bfloat16)
    q = jnp.dot(h, w2, preferred_element_type=jnp.float32) + b2_ref[...]
    o_ref[...] = q[:, : o_ref.shape[-1]].astype(o_ref.dtype)


@functools.partial(jax.jit, static_argnames=("num_actions", "tb"))
def _forward(x, w1p, b1p, w2p, b2p, *, num_actions, tb):
    B, K = x.shape
    H = w1p.shape[1]
    Ap = w2p.shape[1]
    A = num_actions

    grid = (pl.cdiv(B, tb),)
    cost = pl.CostEstimate(
        flops=2 * B * (K * H + H * Ap),
        transcendentals=0,
        bytes_accessed=4 * (B * K + B * A + K * H + H * Ap + H + Ap),
    )

    return pl.pallas_call(
        _mlp_kernel,
        out_shape=jax.ShapeDtypeStruct((B, A), x.dtype),
        grid=grid,
        in_specs=[
            pl.BlockSpec((tb, K), lambda i: (i, 0)),
            pl.BlockSpec((K, H), lambda i: (0, 0)),
            pl.BlockSpec((1, H), lambda i: (0, 0)),
            pl.BlockSpec((H, Ap), lambda i: (0, 0)),
            pl.BlockSpec((1, Ap), lambda i: (0, 0)),
        ],
        out_specs=pl.BlockSpec((tb, A), lambda i: (i, 0)),
        compiler_params=pltpu.CompilerParams(
            dimension_semantics=("parallel",),
            vmem_limit_bytes=96 * 1024 * 1024,
        ),
        cost_estimate=cost,
    )(x, w1p, b1p, w2p, b2p)


def kernel(x, w1p, b1p, w2p, b2p):
    return _forward(x, w1p, b1p, w2p, b2p, num_actions=18, tb=8192)
